# Initial kernel scaffold; baseline (speedup 1.0000x reference)
import jax
import jax.numpy as jnp
from jax.experimental import pallas as pl


def kernel(sent_inputs, tag_inputs, rel_inputs, word_table, tag_table, rel_table):
    raise NotImplementedError("write your pallas kernel here")


# speedup vs baseline: 2.7977x; 2.7977x over previous
import functools

import jax
import jax.numpy as jnp
from jax import lax
from jax.experimental import pallas as pl
from jax.experimental.pallas import tpu as pltpu
from jax.experimental.pallas import tpu_sc as plsc

VOCAB_SIZE = 1000000
TAG_VOCAB = 1000
REL_VOCAB = 1000
WORD_DIM = 64
TAG_DIM = 32
REL_DIM = 32
B = 4096
L = 200
N = B * L

NC = 2
NS = 16
NW = NC * NS
PER_W = N // NW
IDXW = 128
ROWS_W = PER_W // IDXW

R = 4
CH = R * IDXW
NCH = ROWS_W // R

_mesh = plsc.VectorSubcoreMesh(
    core_axis_name="c", subcore_axis_name="s", num_cores=NC, num_subcores=NS
)


@functools.partial(
    pl.kernel,
    out_type=(
        jax.ShapeDtypeStruct((N, WORD_DIM), jnp.float32),
        jax.ShapeDtypeStruct((N, TAG_DIM), jnp.float32),
        jax.ShapeDtypeStruct((N, REL_DIM), jnp.float32),
    ),
    mesh=_mesh,
    scratch_types=(
        pltpu.VMEM((2, R, IDXW), jnp.int32),
        pltpu.VMEM((2, CH, WORD_DIM), jnp.float32),
        pltpu.VMEM((2, R, IDXW), jnp.int32),
        pltpu.VMEM((2, CH, TAG_DIM), jnp.float32),
        pltpu.SemaphoreType.DMA,
        pltpu.SemaphoreType.DMA,
        pltpu.SemaphoreType.DMA,
        pltpu.SemaphoreType.DMA,
        pltpu.SemaphoreType.DMA,
        pltpu.SemaphoreType.DMA,
    ),
    compiler_params=pltpu.CompilerParams(
        needs_layout_passes=False, use_tc_tiling_on_sc=False
    ),
)
def _emb3(sent_hbm, tag_hbm, rel_hbm, wtab_hbm, ttab_hbm, rtab_hbm,
          wout_hbm, tout_hbm, rout_hbm,
          widx_v, wrows_v, sidx_v, srows_v,
          si0, si1, sg0, sg1, so0, so1):
    wid = lax.axis_index("s") * NC + lax.axis_index("c")
    row0 = wid * ROWS_W
    out0 = wid * PER_W
    si = (si0, si1)
    sg = (sg0, sg1)
    so = (so0, so1)

    def run_phase(idx_hbm, tab_hbm, out_hbm, idx_v, rows_v, fix):
        def idx_src(c):
            return idx_hbm.at[pl.ds(row0 + c * R, R)]

        def out_dst(c):
            return out_hbm.at[pl.ds(out0 + c * CH, CH)]

        def fire_idx(b, c):
            pltpu.async_copy(idx_src(c), idx_v.at[b], si[b])

        def wait_idx(b):
            pltpu.make_async_copy(idx_src(0), idx_v.at[b], si[b]).wait()

        def fire_g(b):
            for j in range(R):
                pltpu.async_copy(
                    tab_hbm.at[idx_v.at[b].at[j]],
                    rows_v.at[b].at[pl.ds(j * IDXW, IDXW)],
                    sg[b],
                )

        def wait_g(b):
            for j in range(R):
                pltpu.make_async_copy(
                    tab_hbm.at[idx_v.at[b].at[j]],
                    rows_v.at[b].at[pl.ds(j * IDXW, IDXW)],
                    sg[b],
                ).wait()

        def fire_w(b, c):
            pltpu.async_copy(rows_v.at[b], out_dst(c), so[b])

        def wait_w(b):
            pltpu.make_async_copy(rows_v.at[b], out_dst(0), so[b]).wait()

        def zero_fix(b):
            def group(g, _):
                j = g // (IDXW // 16)
                o = (g % (IDXW // 16)) * 16
                iv = idx_v[b, j, pl.ds(o, 16)]
                mask = iv == 0
                nz = plsc.all_reduce_population_count(mask)

                @pl.when(nz[0] > 0)
                def _():
                    rowi = g * 16 + lax.iota(jnp.int32, 16)
                    zz = jnp.zeros((16,), jnp.float32)
                    for col in range(WORD_DIM):
                        plsc.store_scatter(
                            rows_v.at[b],
                            [rowi, jnp.full((16,), col, jnp.int32)],
                            zz,
                            mask=mask,
                        )
                return 0

            lax.fori_loop(0, CH // 16, group, 0)

        pltpu.sync_copy(idx_src(0), idx_v.at[0])
        fire_g(0)
        fire_idx(1, 1)

        def body(k, _):
            c0 = 2 * k
            c1 = c0 + 1
            wait_g(0)
            wait_idx(1)

            @pl.when(k > 0)
            def _():
                wait_w(1)

            fire_g(1)
            if fix:
                zero_fix(0)

            @pl.when(c0 + 2 < NCH)
            def _():
                fire_idx(0, c0 + 2)

            fire_w(0, c0)

            wait_g(1)

            @pl.when(c1 + 1 < NCH)
            def _():
                wait_idx(0)
                wait_w(0)
                fire_g(0)

            if fix:
                zero_fix(1)

            @pl.when(c1 + 2 < NCH)
            def _():
                fire_idx(1, c1 + 2)

            fire_w(1, c1)
            return 0

        lax.fori_loop(0, NCH // 2, body, 0)
        wait_w(0)
        wait_w(1)

    run_phase(sent_hbm, wtab_hbm, wout_hbm, widx_v, wrows_v, fix=True)
    run_phase(tag_hbm, ttab_hbm, tout_hbm, sidx_v, srows_v, fix=False)
    run_phase(rel_hbm, rtab_hbm, rout_hbm, sidx_v, srows_v, fix=False)


def kernel(sent_inputs, tag_inputs, rel_inputs, word_table, tag_table, rel_table):
    sent = sent_inputs.reshape(N // IDXW, IDXW).astype(jnp.int32)
    tag = tag_inputs.reshape(N // IDXW, IDXW).astype(jnp.int32)
    rel = rel_inputs.reshape(N // IDXW, IDXW).astype(jnp.int32)
    wout, tout, rout = _emb3(sent, tag, rel, word_table, tag_table, rel_table)
    return (
        wout.reshape(B, 1, L, WORD_DIM),
        tout.reshape(B, 1, L, TAG_DIM),
        rout.reshape(B, 1, L, REL_DIM),
    )
